# Initial kernel scaffold; baseline (speedup 1.0000x reference)
import functools
import math

import jax
import jax.numpy as jnp
from jax import lax
from jax.experimental import pallas as pl
from jax.experimental.pallas import tpu as pltpu
from jax.experimental.pallas import tpu_sc as plsc

_DIM = 32
_EMB_SCALE = math.sqrt(float(_DIM))
_NC, _NS, _L = 2, 16, 16
_NW = _NC * _NS
_CH = 128
_NBUF = 8


def _make_kernel(nch):
  mesh = plsc.VectorSubcoreMesh(core_axis_name="c", subcore_axis_name="s")

  @functools.partial(
      pl.kernel,
      out_type=jax.ShapeDtypeStruct((_NW, nch, _CH, _DIM), jnp.float32),
      mesh=mesh,
      scratch_types=[
          pltpu.VMEM((nch, _CH), jnp.int32),
          pltpu.VMEM((_NBUF, _CH, _DIM), jnp.float32),
          [pltpu.SemaphoreType.DMA] * _NBUF,
          [pltpu.SemaphoreType.DMA] * _NBUF,
      ],
  )
  def body(x_hbm, table_hbm, out_hbm, idx_v, rows_v, gsems, osems):
    wid = lax.axis_index("s") * _NC + lax.axis_index("c")
    pltpu.sync_copy(x_hbm.at[wid], idx_v)

    @pl.loop(0, nch, step=_NBUF)
    def _group(g):
      for b in range(_NBUF):
        @pl.when(g > 0)
        def _drain():
          pltpu.make_async_copy(
              rows_v.at[b], out_hbm.at[wid, g - _NBUF + b], osems[b]).wait()
        pltpu.async_copy(
            table_hbm.at[idx_v.at[g + b]], rows_v.at[b], gsems[b])
      for b in range(_NBUF):
        c = g + b
        pltpu.make_async_copy(
            table_hbm.at[idx_v.at[c]], rows_v.at[b], gsems[b]).wait()

        @pl.loop(0, _CH, unroll=4)
        def _scale(r):
          for j in range(_DIM // _L):
            sl = (b, r, pl.ds(j * _L, _L))
            rows_v[sl] = rows_v[sl] * _EMB_SCALE

        pltpu.async_copy(rows_v.at[b], out_hbm.at[wid, c], osems[b])

    for b in range(_NBUF):
      pltpu.make_async_copy(
          rows_v.at[b], out_hbm.at[wid, nch - _NBUF + b], osems[b]).wait()

  return body


def kernel(x, table):
  bsz, seq = x.shape
  tot = bsz * seq
  nch = tot // (_NW * _CH)
  xr = x.astype(jnp.int32).reshape(_NW, nch, _CH)
  out = _make_kernel(nch)(xr, table)
  return out.reshape(bsz, seq, _DIM)


# speedup vs baseline: 1.4760x; 1.4760x over previous
import functools
import math

import jax
import jax.numpy as jnp
from jax import lax
from jax.experimental import pallas as pl
from jax.experimental.pallas import tpu as pltpu
from jax.experimental.pallas import tpu_sc as plsc

_DIM = 32
_EMB_SCALE = math.sqrt(float(_DIM))
_NC, _NS, _L = 2, 16, 16
_NW = _NC * _NS
_CH = 128
_NBUF = 8


def _make_kernel(nch):
  mesh = plsc.VectorSubcoreMesh(core_axis_name="c", subcore_axis_name="s")

  @functools.partial(
      pl.kernel,
      out_type=jax.ShapeDtypeStruct((_NW, nch, _CH, _DIM), jnp.float32),
      mesh=mesh,
      compiler_params=pltpu.CompilerParams(use_tc_tiling_on_sc=False),
      scratch_types=[
          pltpu.VMEM((nch, _CH), jnp.int32),
          pltpu.VMEM((_NBUF, _CH, _DIM), jnp.float32),
          [pltpu.SemaphoreType.DMA] * _NBUF,
          [pltpu.SemaphoreType.DMA] * _NBUF,
      ],
  )
  def body(x_hbm, table_hbm, out_hbm, idx_v, rows_v, gsems, osems):
    wid = lax.axis_index("s") * _NC + lax.axis_index("c")
    pltpu.sync_copy(x_hbm.at[wid], idx_v)

    @pl.loop(0, nch, step=_NBUF)
    def _group(g):
      for b in range(_NBUF):
        @pl.when(g > 0)
        def _drain():
          pltpu.make_async_copy(
              rows_v.at[b], out_hbm.at[wid, g - _NBUF + b], osems[b]).wait()
        pltpu.async_copy(
            table_hbm.at[idx_v.at[g + b]], rows_v.at[b], gsems[b])
      for b in range(_NBUF):
        c = g + b
        pltpu.make_async_copy(
            table_hbm.at[idx_v.at[c]], rows_v.at[b], gsems[b]).wait()

        @pl.loop(0, _CH, unroll=4)
        def _scale(r):
          for j in range(_DIM // _L):
            sl = (b, r, pl.ds(j * _L, _L))
            rows_v[sl] = rows_v[sl] * _EMB_SCALE

        pltpu.async_copy(rows_v.at[b], out_hbm.at[wid, c], osems[b])

    for b in range(_NBUF):
      pltpu.make_async_copy(
          rows_v.at[b], out_hbm.at[wid, nch - _NBUF + b], osems[b]).wait()

  return body


def kernel(x, table):
  bsz, seq = x.shape
  tot = bsz * seq
  nch = tot // (_NW * _CH)
  xr = x.astype(jnp.int32).reshape(_NW, nch, _CH)
  out = _make_kernel(nch)(xr, table)
  return out.reshape(bsz, seq, _DIM)
